# baseline (speedup 1.0000x reference)
import functools

import jax
import jax.numpy as jnp
from jax.experimental import pallas as pl
from jax.experimental.pallas import tpu as pltpu
from jax.experimental.pallas import tpu_sc as plsc


_NSUB = 16


def _sc_gather(c_table, c_idx, m_table, m_idx):
    nc = c_idx.shape[0]
    nm = m_idx.shape[0]
    wc = nc // _NSUB
    wm = nm // _NSUB
    dim = c_table.shape[1]
    mesh = plsc.VectorSubcoreMesh(core_axis_name="c", subcore_axis_name="s")

    @pl.kernel(
        out_type=(
            jax.ShapeDtypeStruct((nc, dim), c_table.dtype),
            jax.ShapeDtypeStruct((nm, dim), m_table.dtype),
        ),
        mesh=mesh,
        scratch_types=[
            pltpu.VMEM((max(wc, wm),), jnp.int32),
            pltpu.VMEM((max(wc, wm), dim), jnp.float32),
            pltpu.SemaphoreType.DMA,
        ],
    )
    def gather_kernel(c_hbm, ci_hbm, m_hbm, mi_hbm, o_dia, o_med,
                      idxb, rowb, sem):
        core = jax.lax.axis_index("c")
        sub = jax.lax.axis_index("s")

        @pl.when(core == 0)
        def _():
            base = sub * wc
            pltpu.async_copy(ci_hbm.at[pl.ds(base, wc)], idxb.at[:wc],
                             sem).wait()
            pltpu.sync_copy(c_hbm.at[idxb.at[:wc]], rowb.at[:wc])
            pltpu.async_copy(rowb.at[:wc], o_dia.at[pl.ds(base, wc)],
                             sem).wait()

        @pl.when(core == 1)
        def _():
            base = sub * wm
            pltpu.async_copy(mi_hbm.at[pl.ds(base, wm)], idxb.at[:wm],
                             sem).wait()
            pltpu.sync_copy(m_hbm.at[idxb.at[:wm]], rowb.at[:wm])
            pltpu.async_copy(rowb.at[:wm], o_med.at[pl.ds(base, wm)],
                             sem).wait()

    return gather_kernel(c_table, c_idx, m_table, m_idx)


def _tc_body(nc, nm, dia_ref, med_ref, hat, w1, b1, w2, att2, b2, wl,
             o1, o2):
    f32 = jnp.float32
    c = w2.shape[1]
    dia = dia_ref[...][:nc]
    med = med_ref[...][:nm]
    xd = jnp.dot(dia, w2[...], preferred_element_type=f32)
    xm = jnp.dot(med, w2[...], preferred_element_type=f32)
    he = jnp.dot(hat[...], w2[...], preferred_element_type=f32)

    att = att2[...]
    an = att[:c][None, :]
    ae = att[c:][None, :]
    b1v = b1[...][None, :]
    b2v = b2[...][None, :]
    wl_t = wl[...][:c]
    wl_b = wl[...][c:]
    v = jnp.sum(he * ae, axis=1, keepdims=True)
    ud = jnp.sum(xd * an, axis=1, keepdims=True)
    um = jnp.sum(xm * an, axis=1)

    lrelu = lambda x: jnp.where(x >= 0, x, 0.2 * x)
    a_dis = lrelu(ud + v)
    amat = lrelu(v + um[None, :])
    a_max = jnp.maximum(jnp.max(amat, axis=1, keepdims=True), a_dis)
    emat = jnp.exp(amat - a_max)
    p = jnp.exp(a_dis - a_max)
    ssum = jnp.sum(emat, axis=1, keepdims=True)
    denom = p + ssum + 1e-16
    g = jnp.dot(emat, xm, preferred_element_type=f32)
    ef = (p * xd + g) / denom * (1.0 / (nm + 1))
    sum1 = jnp.sum((p / denom) * ef, axis=0)[None, :]
    sum2 = jnp.sum((ssum / denom) * ef, axis=0)[None, :]

    sum_dia = jnp.sum(dia, axis=0)[None, :]
    sum_med = jnp.sum(med, axis=0)[None, :]
    t1 = jnp.dot(sum_dia, w1[...], preferred_element_type=f32) + nc * b1v
    t2 = jnp.dot(sum_med, w1[...], preferred_element_type=f32) + nm * b1v

    r1 = sum1 + nc * b2v
    r2 = sum2 * (1.0 / nc) + nm * b2v
    o1[...] = (jnp.dot(r1, wl_t, preferred_element_type=f32)
               + jnp.dot(t1, wl_b, preferred_element_type=f32))
    o2[...] = (jnp.dot(r2, wl_t, preferred_element_type=f32)
               + jnp.dot(t2, wl_b, preferred_element_type=f32))


def kernel(c_it, medicine_it, c_embeddings, m_embeddings, W1, b1, W2, att2,
           b2, Wl, hyperedge_attr):
    nc = c_it.shape[0]
    nm = medicine_it.shape[0]
    c = W2.shape[1]

    w = 64
    nc_pad = -(-nc // w) * w
    ci = jnp.pad(c_it.astype(jnp.int32), (0, nc_pad - nc))
    mi = medicine_it.astype(jnp.int32)
    dia, med = _sc_gather(c_embeddings, ci, m_embeddings, mi)

    i1, i2 = pl.pallas_call(
        functools.partial(_tc_body, nc, nm),
        out_shape=(
            jax.ShapeDtypeStruct((1, c), jnp.float32),
            jax.ShapeDtypeStruct((1, c), jnp.float32),
        ),
    )(dia, med, hyperedge_attr, W1, b1, W2, att2, b2, Wl)

    return i1.reshape(1, 1, c), i2.reshape(1, 1, c)


# speedup vs baseline: 1.0051x; 1.0051x over previous
import functools

import jax
import jax.numpy as jnp
from jax.experimental import pallas as pl
from jax.experimental.pallas import tpu as pltpu
from jax.experimental.pallas import tpu_sc as plsc


_NSUB = 16


def _sc_gather(c_table, c_idx, m_table, m_idx):
    nc = c_idx.shape[0]
    nm = m_idx.shape[0]
    wc = nc // _NSUB
    wm = nm // _NSUB
    dim = c_table.shape[1]
    mesh = plsc.VectorSubcoreMesh(core_axis_name="c", subcore_axis_name="s")

    @pl.kernel(
        out_type=(
            jax.ShapeDtypeStruct((nc, dim), c_table.dtype),
            jax.ShapeDtypeStruct((nm, dim), m_table.dtype),
        ),
        mesh=mesh,
        scratch_types=[
            pltpu.VMEM((max(wc, wm),), jnp.int32),
            pltpu.VMEM((max(wc, wm), dim), jnp.float32),
            pltpu.SemaphoreType.DMA,
        ],
    )
    def gather_kernel(c_hbm, ci_hbm, m_hbm, mi_hbm, o_dia, o_med,
                      idxb, rowb, sem):
        core = jax.lax.axis_index("c")
        sub = jax.lax.axis_index("s")

        @pl.when(core == 0)
        def _():
            base = sub * wc
            pltpu.async_copy(ci_hbm.at[pl.ds(base, wc)], idxb.at[:wc],
                             sem).wait()
            pltpu.sync_copy(c_hbm.at[idxb.at[:wc]], rowb.at[:wc])
            pltpu.async_copy(rowb.at[:wc], o_dia.at[pl.ds(base, wc)],
                             sem).wait()

        @pl.when(core == 1)
        def _():
            base = sub * wm
            pltpu.async_copy(mi_hbm.at[pl.ds(base, wm)], idxb.at[:wm],
                             sem).wait()
            pltpu.sync_copy(m_hbm.at[idxb.at[:wm]], rowb.at[:wm])
            pltpu.async_copy(rowb.at[:wm], o_med.at[pl.ds(base, wm)],
                             sem).wait()

    return gather_kernel(c_table, c_idx, m_table, m_idx)


def _tc_body(nc, nm, dia_ref, med_ref, hat, w1, b1, w2, att2, b2, wl,
             o1, o2):
    f32 = jnp.float32
    c = w2.shape[1]
    dia = dia_ref[...][:nc]
    med = med_ref[...][:nm]
    xd = jnp.dot(dia, w2[...], preferred_element_type=f32)
    xm = jnp.dot(med, w2[...], preferred_element_type=f32)
    he = jnp.dot(hat[...], w2[...], preferred_element_type=f32)

    att = att2[...]
    an = att[:c][None, :]
    ae = att[c:][None, :]
    b1v = b1[...][None, :]
    b2v = b2[...][None, :]
    wl_t = wl[...][:c]
    wl_b = wl[...][c:]
    v = jnp.sum(he * ae, axis=1, keepdims=True)
    ud = jnp.sum(xd * an, axis=1, keepdims=True)
    um = jnp.sum(xm * an, axis=1)

    lrelu = lambda x: jnp.maximum(x, 0.2 * x)
    a_dis = lrelu(ud + v)
    amat = lrelu(v + um[None, :])
    a_max = jnp.maximum(jnp.max(amat, axis=1, keepdims=True), a_dis)
    emat = jnp.exp(amat - a_max)
    p = jnp.exp(a_dis - a_max)
    ssum = jnp.sum(emat, axis=1, keepdims=True)
    denom = p + ssum + 1e-16
    g = jnp.dot(emat, xm, preferred_element_type=f32)
    ef = (p * xd + g) / denom * (1.0 / (nm + 1))
    sum1 = jnp.sum((p / denom) * ef, axis=0)[None, :]
    sum2 = jnp.sum((ssum / denom) * ef, axis=0)[None, :]

    sum_dia = jnp.sum(dia, axis=0)[None, :]
    sum_med = jnp.sum(med, axis=0)[None, :]
    t1 = jnp.dot(sum_dia, w1[...], preferred_element_type=f32) + nc * b1v
    t2 = jnp.dot(sum_med, w1[...], preferred_element_type=f32) + nm * b1v

    r1 = sum1 + nc * b2v
    r2 = sum2 * (1.0 / nc) + nm * b2v
    o1[...] = (jnp.dot(r1, wl_t, preferred_element_type=f32)
               + jnp.dot(t1, wl_b, preferred_element_type=f32))
    o2[...] = (jnp.dot(r2, wl_t, preferred_element_type=f32)
               + jnp.dot(t2, wl_b, preferred_element_type=f32))


def kernel(c_it, medicine_it, c_embeddings, m_embeddings, W1, b1, W2, att2,
           b2, Wl, hyperedge_attr):
    nc = c_it.shape[0]
    nm = medicine_it.shape[0]
    c = W2.shape[1]

    w = 64
    nc_pad = -(-nc // w) * w
    ci = jnp.pad(c_it.astype(jnp.int32), (0, nc_pad - nc))
    mi = medicine_it.astype(jnp.int32)
    dia, med = _sc_gather(c_embeddings, ci, m_embeddings, mi)

    i1, i2 = pl.pallas_call(
        functools.partial(_tc_body, nc, nm),
        out_shape=(
            jax.ShapeDtypeStruct((1, c), jnp.float32),
            jax.ShapeDtypeStruct((1, c), jnp.float32),
        ),
    )(dia, med, hyperedge_attr, W1, b1, W2, att2, b2, Wl)

    return i1.reshape(1, 1, c), i2.reshape(1, 1, c)


# speedup vs baseline: 1.0407x; 1.0354x over previous
import functools

import jax
import jax.numpy as jnp
from jax.experimental import pallas as pl
from jax.experimental.pallas import tpu as pltpu
from jax.experimental.pallas import tpu_sc as plsc


_NSUB = 16


def _sc_gather(c_table, c_idx, m_table, m_idx):
    nc = c_idx.shape[0]
    nm = m_idx.shape[0]
    wc = (-(-nc // _NSUB) + 15) // 16 * 16
    full_c = nc // wc
    tail_c = nc - full_c * wc
    wm = nm // _NSUB
    dim = c_table.shape[1]
    mesh = plsc.VectorSubcoreMesh(core_axis_name="c", subcore_axis_name="s")

    @pl.kernel(
        out_type=(
            jax.ShapeDtypeStruct((nc, dim), c_table.dtype),
            jax.ShapeDtypeStruct((nm, dim), m_table.dtype),
        ),
        mesh=mesh,
        scratch_types=[
            pltpu.VMEM((max(wc, wm),), jnp.int32),
            pltpu.VMEM((max(wc, wm), dim), jnp.float32),
            pltpu.SemaphoreType.DMA,
        ],
    )
    def gather_kernel(c_hbm, ci_hbm, m_hbm, mi_hbm, o_dia, o_med,
                      idxb, rowb, sem):
        core = jax.lax.axis_index("c")
        sub = jax.lax.axis_index("s")

        @pl.when((core == 0) & (sub < full_c))
        def _():
            base = sub * wc
            pltpu.async_copy(ci_hbm.at[pl.ds(base, wc)], idxb.at[:wc],
                             sem).wait()
            pltpu.sync_copy(c_hbm.at[idxb.at[:wc]], rowb.at[:wc])
            pltpu.async_copy(rowb.at[:wc], o_dia.at[pl.ds(base, wc)],
                             sem).wait()

        if tail_c:
            @pl.when((core == 0) & (sub == full_c))
            def _():
                base = full_c * wc
                pltpu.async_copy(ci_hbm.at[pl.ds(base, tail_c)],
                                 idxb.at[:tail_c], sem).wait()
                pltpu.sync_copy(c_hbm.at[idxb.at[:tail_c]],
                                rowb.at[:tail_c])
                pltpu.async_copy(rowb.at[:tail_c],
                                 o_dia.at[pl.ds(base, tail_c)], sem).wait()

        @pl.when(core == 1)
        def _():
            base = sub * wm
            pltpu.async_copy(mi_hbm.at[pl.ds(base, wm)], idxb.at[:wm],
                             sem).wait()
            pltpu.sync_copy(m_hbm.at[idxb.at[:wm]], rowb.at[:wm])
            pltpu.async_copy(rowb.at[:wm], o_med.at[pl.ds(base, wm)],
                             sem).wait()

    return gather_kernel(c_table, c_idx, m_table, m_idx)


def _tc_body(nc, nm, dia_ref, med_ref, hat, w1, b1, w2, att2, b2, wl,
             o1, o2):
    f32 = jnp.float32
    c = w2.shape[1]
    dia = dia_ref[...][:nc]
    med = med_ref[...][:nm]
    xd = jnp.dot(dia, w2[...], preferred_element_type=f32)
    xm = jnp.dot(med, w2[...], preferred_element_type=f32)
    he = jnp.dot(hat[...], w2[...], preferred_element_type=f32)

    att = att2[...]
    an = att[:c][None, :]
    ae = att[c:][None, :]
    b1v = b1[...][None, :]
    b2v = b2[...][None, :]
    wl_t = wl[...][:c]
    wl_b = wl[...][c:]
    v = jnp.sum(he * ae, axis=1, keepdims=True)
    ud = jnp.sum(xd * an, axis=1, keepdims=True)
    um = jnp.sum(xm * an, axis=1)

    lrelu = lambda x: jnp.maximum(x, 0.2 * x)
    a_dis = lrelu(ud + v)
    amat = lrelu(v + um[None, :])
    a_max = jnp.maximum(jnp.max(amat, axis=1, keepdims=True), a_dis)
    emat = jnp.exp(amat - a_max)
    p = jnp.exp(a_dis - a_max)
    ssum = jnp.sum(emat, axis=1, keepdims=True)
    denom = p + ssum + 1e-16
    g = jnp.dot(emat, xm, preferred_element_type=f32)
    ef = (p * xd + g) / denom * (1.0 / (nm + 1))
    sum1 = jnp.sum((p / denom) * ef, axis=0)[None, :]
    sum2 = jnp.sum((ssum / denom) * ef, axis=0)[None, :]

    sum_dia = jnp.sum(dia, axis=0)[None, :]
    sum_med = jnp.sum(med, axis=0)[None, :]
    t1 = jnp.dot(sum_dia, w1[...], preferred_element_type=f32) + nc * b1v
    t2 = jnp.dot(sum_med, w1[...], preferred_element_type=f32) + nm * b1v

    r1 = sum1 + nc * b2v
    r2 = sum2 * (1.0 / nc) + nm * b2v
    o1[...] = (jnp.dot(r1, wl_t, preferred_element_type=f32)
               + jnp.dot(t1, wl_b, preferred_element_type=f32))
    o2[...] = (jnp.dot(r2, wl_t, preferred_element_type=f32)
               + jnp.dot(t2, wl_b, preferred_element_type=f32))


def kernel(c_it, medicine_it, c_embeddings, m_embeddings, W1, b1, W2, att2,
           b2, Wl, hyperedge_attr):
    nc = c_it.shape[0]
    nm = medicine_it.shape[0]
    c = W2.shape[1]

    ci = c_it.astype(jnp.int32)
    mi = medicine_it.astype(jnp.int32)
    dia, med = _sc_gather(c_embeddings, ci, m_embeddings, mi)

    i1, i2 = pl.pallas_call(
        functools.partial(_tc_body, nc, nm),
        out_shape=(
            jax.ShapeDtypeStruct((1, c), jnp.float32),
            jax.ShapeDtypeStruct((1, c), jnp.float32),
        ),
    )(dia, med, hyperedge_attr, W1, b1, W2, att2, b2, Wl)

    return i1.reshape(1, 1, c), i2.reshape(1, 1, c)
